# Initial kernel scaffold; baseline (speedup 1.0000x reference)
import jax
import jax.numpy as jnp
from jax.experimental import pallas as pl


def kernel(x, edge_index, W1, b1, W2, b2, temp):
    raise NotImplementedError("write your pallas kernel here")


# speedup vs baseline: 11.4437x; 11.4437x over previous
import functools

import jax
import jax.numpy as jnp
from jax import lax
from jax.experimental import pallas as pl
from jax.experimental.pallas import tpu as pltpu
from jax.experimental.pallas import tpu_sc as plsc

NS = 16
NC = 2
LANES = 16
CHUNK = 128


def _rsqrt_newton(v):
    i = lax.bitcast_convert_type(v, jnp.int32)
    i = jnp.int32(0x5F3759DF) - (i >> 1)
    y = lax.bitcast_convert_type(i, jnp.float32)
    half = v * jnp.float32(-0.5)
    for _ in range(4):
        y = y * (jnp.float32(1.5) + half * y * y)
    return jnp.where(v > jnp.float32(0.5), y, jnp.float32(0.0))


def _dense_body(x_ref, w1_ref, b1_ref, w2_ref, b2_ref, h0_ref, h1_ref):
    hb = jnp.dot(x_ref[...], w1_ref[...], preferred_element_type=jnp.float32)
    hb = jnp.maximum(hb + b1_ref[...], 0.0)
    ob = jnp.dot(hb, w2_ref[...], preferred_element_type=jnp.float32)
    ob = ob + b2_ref[...]
    ch = ob.shape[1] // 2
    h0_ref[...] = ob[:, :ch]
    h1_ref[...] = ob[:, ch:]


def _softmax_body(a_ref, b_ref, o_ref):
    xb = jnp.concatenate([a_ref[...], b_ref[...]], axis=1)
    m = jnp.max(xb, axis=1, keepdims=True)
    ex = jnp.exp(xb - m)
    lse = jnp.log(jnp.sum(ex, axis=1, keepdims=True)) + m
    o_ref[...] = xb - lse


def _make_sc_kernel(NP, EPT, K, CHW):
    R = NP // NS
    CH = EPT // CHUNK
    mesh = plsc.VectorSubcoreMesh(core_axis_name="c", subcore_axis_name="s")

    def body(h2, srcs, dsts, temp16, hid_out,
             za, zb, S, src_v, dst_v, rows0, d1_v, hid_v, s_v, z_v, t_v):
        c = lax.axis_index("c")
        s = lax.axis_index("s")
        r0 = s * R
        base = c * NP + r0

        pltpu.sync_copy(temp16, t_v)
        pltpu.sync_copy(srcs.at[s], src_v)
        pltpu.sync_copy(dsts.at[s], dst_v)
        off = jnp.full((LANES,), c * NP, jnp.int32)

        def adj(i, carry):
            a = i // (CHUNK // LANES)
            b = (i % (CHUNK // LANES)) * LANES
            src_v[a, pl.ds(b, LANES)] = src_v[a, pl.ds(b, LANES)] + off
            return carry

        lax.fori_loop(0, CH * (CHUNK // LANES), adj, 0)

        zf32 = jnp.zeros((LANES,), jnp.float32)
        ones = jnp.full((LANES,), 1.0, jnp.float32)

        def zrow(i, carry):
            for l in range(0, CHW, LANES):
                s_v[i, pl.ds(l, LANES)] = zf32
            return carry

        lax.fori_loop(0, R, zrow, 0)
        pltpu.sync_copy(s_v, S.at[pl.ds(r0, R)])

        def orow(i, carry):
            for l in range(0, CHW, LANES):
                rows0[i, pl.ds(l, LANES)] = ones
            return carry

        lax.fori_loop(0, CHUNK, orow, 0)
        plsc.subcore_barrier()

        def degc(j, carry):
            pltpu.sync_copy(rows0, S.at[dst_v.at[j]], add=True)
            return carry

        lax.fori_loop(0, CH, degc, 0)
        plsc.subcore_barrier()

        pltpu.sync_copy(S.at[pl.ds(r0, R)], s_v)

        def drow(i, carry):
            for l in range(0, CHW, LANES):
                v = s_v[i, pl.ds(l, LANES)]
                d1_v[i, pl.ds(l, LANES)] = _rsqrt_newton(v)
                s_v[i, pl.ds(l, LANES)] = zf32
            return carry

        lax.fori_loop(0, R, drow, 0)
        pltpu.sync_copy(s_v, S.at[pl.ds(r0, R)])

        t0 = t_v[0, pl.ds(0, LANES)]
        pltpu.sync_copy(h2.at[pl.ds(base, R)], z_v)

        def irow(i, carry):
            for l in range(0, CHW, LANES):
                h = z_v[i, pl.ds(l, LANES)]
                hid_v[i, pl.ds(l, LANES)] = t0 * h
                z_v[i, pl.ds(l, LANES)] = d1_v[i, pl.ds(l, LANES)] * h
            return carry

        lax.fori_loop(0, R, irow, 0)
        pltpu.sync_copy(z_v, za.at[pl.ds(base, R)])
        plsc.subcore_barrier()

        for k in range(K):
            zread, zwrite = (za, zb) if k % 2 == 0 else (zb, za)

            def gsc(j, carry):
                pltpu.sync_copy(zread.at[src_v.at[j]], rows0)
                pltpu.sync_copy(rows0, S.at[dst_v.at[j]], add=True)
                return carry

            lax.fori_loop(0, CH, gsc, 0)
            plsc.subcore_barrier()

            pltpu.sync_copy(S.at[pl.ds(r0, R)], s_v)
            tk = t_v[k + 1, pl.ds(0, LANES)]
            last = k == K - 1

            def brow(i, carry):
                for l in range(0, CHW, LANES):
                    sv = s_v[i, pl.ds(l, LANES)]
                    d = d1_v[i, pl.ds(l, LANES)]
                    m = d * sv
                    hid_v[i, pl.ds(l, LANES)] = hid_v[i, pl.ds(l, LANES)] + tk * m
                    if not last:
                        z_v[i, pl.ds(l, LANES)] = d * m
                        s_v[i, pl.ds(l, LANES)] = zf32
                return carry

            lax.fori_loop(0, R, brow, 0)
            if not last:
                pltpu.sync_copy(z_v, zwrite.at[pl.ds(base, R)])
                pltpu.sync_copy(s_v, S.at[pl.ds(r0, R)])
                plsc.subcore_barrier()

        pltpu.sync_copy(hid_v, hid_out.at[pl.ds(base, R)])

    return pl.kernel(
        body,
        out_type=jax.ShapeDtypeStruct((2 * NP, CHW), jnp.float32),
        mesh=mesh,
        compiler_params=pltpu.CompilerParams(use_tc_tiling_on_sc=False),
        scratch_types=[
            pltpu.HBM((2 * NP, CHW), jnp.float32),
            pltpu.HBM((2 * NP, CHW), jnp.float32),
            pltpu.VMEM_SHARED((NP, CHW), jnp.float32),
            pltpu.VMEM((EPT // CHUNK, CHUNK), jnp.int32),
            pltpu.VMEM((EPT // CHUNK, CHUNK), jnp.int32),
            pltpu.VMEM((CHUNK, CHW), jnp.float32),
            pltpu.VMEM((NP // NS, CHW), jnp.float32),
            pltpu.VMEM((NP // NS, CHW), jnp.float32),
            pltpu.VMEM((NP // NS, CHW), jnp.float32),
            pltpu.VMEM((NP // NS, CHW), jnp.float32),
            pltpu.VMEM((LANES, LANES), jnp.float32),
        ],
    )


def kernel(x, edge_index, W1, b1, W2, b2, temp):
    N, F_IN = x.shape
    C = W2.shape[1]
    CHW = C // 2
    K = temp.shape[0] - 1
    E = edge_index.shape[1]

    NP = -(-(N + 2) // (NS * 8)) * (NS * 8)
    EPAD = NS * CHUNK
    EP = -(-(E + N) // EPAD) * EPAD
    EPT = EP // NS

    loop = jnp.arange(N, dtype=jnp.int32)
    pad = jnp.full((EP - E - N,), NP - 1, jnp.int32)
    srcs = jnp.concatenate([edge_index[0].astype(jnp.int32), loop, pad])
    dsts = jnp.concatenate([edge_index[1].astype(jnp.int32), loop, pad])
    srcs = srcs.reshape(NS, EPT // CHUNK, CHUNK)
    dsts = dsts.reshape(NS, EPT // CHUNK, CHUNK)
    x_pad = jnp.zeros((NP, F_IN), x.dtype).at[:N].set(x)
    temp16 = jnp.zeros((LANES, LANES), jnp.float32).at[: K + 1, :].set(
        temp.astype(jnp.float32)[:, None])

    BM = NP
    for cand in (1264, 2504, 1252, 8):
        if NP % cand == 0 and cand % 8 == 0:
            BM = cand
            break
    grid = NP // BM
    HID = W1.shape[1]
    h0, h1 = pl.pallas_call(
        _dense_body,
        grid=(grid,),
        in_specs=[
            pl.BlockSpec((BM, F_IN), lambda i: (i, 0)),
            pl.BlockSpec((F_IN, HID), lambda i: (0, 0)),
            pl.BlockSpec((1, HID), lambda i: (0, 0)),
            pl.BlockSpec((HID, C), lambda i: (0, 0)),
            pl.BlockSpec((1, C), lambda i: (0, 0)),
        ],
        out_specs=[
            pl.BlockSpec((BM, CHW), lambda i: (i, 0)),
            pl.BlockSpec((BM, CHW), lambda i: (i, 0)),
        ],
        out_shape=[
            jax.ShapeDtypeStruct((NP, CHW), jnp.float32),
            jax.ShapeDtypeStruct((NP, CHW), jnp.float32),
        ],
    )(x_pad, W1, b1.reshape(1, HID), W2, b2.reshape(1, C))
    h2 = jnp.concatenate([h0, h1], axis=0)

    sc = _make_sc_kernel(NP, EPT, K, CHW)
    hid = sc(h2, srcs, dsts, temp16)

    out = pl.pallas_call(
        _softmax_body,
        grid=(grid,),
        in_specs=[
            pl.BlockSpec((BM, CHW), lambda i: (i, 0)),
            pl.BlockSpec((BM, CHW), lambda i: (i + NP // BM, 0)),
        ],
        out_specs=pl.BlockSpec((BM, C), lambda i: (i, 0)),
        out_shape=jax.ShapeDtypeStruct((NP, C), jnp.float32),
    )(hid, hid)
    return out[:N]
